# baseline (speedup 1.0000x reference)
import functools

import jax
import jax.numpy as jnp
from jax import lax
from jax.experimental import pallas as pl
from jax.experimental.pallas import tpu as pltpu
from jax.experimental.pallas import tpu_sc as plsc

_EMB = 64
_N = 4096 * 50

_info = plsc.get_sparse_core_info()
_NC = _info.num_cores
_NS = _info.num_subcores
_NW = _NC * _NS
_BPW = _N // _NW
_CHUNK = 128
_NCH = _BPW // _CHUNK

_mesh = plsc.VectorSubcoreMesh(core_axis_name="c", subcore_axis_name="s")


@functools.partial(
    pl.kernel,
    mesh=_mesh,
    compiler_params=pltpu.CompilerParams(use_tc_tiling_on_sc=False),
    out_type=jax.ShapeDtypeStruct((_N, _EMB), jnp.float32),
    scratch_types=[
        pltpu.VMEM((_NCH, _CHUNK), jnp.int32),
        pltpu.VMEM((_CHUNK, _EMB), jnp.float32),
        pltpu.SemaphoreType.DMA,
    ],
)
def _embed_gather(idx_hbm, table_hbm, out_hbm, idx_v, rows, sem):
    wid = lax.axis_index("s") * _NC + lax.axis_index("c")
    base = wid * _BPW
    pltpu.sync_copy(idx_hbm.at[wid], idx_v)

    def body(c, carry):
        pltpu.async_copy(table_hbm.at[idx_v.at[c]], rows, sem).wait()
        pltpu.sync_copy(rows, out_hbm.at[pl.ds(base + c * _CHUNK, _CHUNK)])
        return carry

    lax.fori_loop(0, _NCH, body, 0)


def kernel(token_ids, weight):
    b, l = token_ids.shape
    idx = token_ids.astype(jnp.int32).reshape(_NW, _NCH, _CHUNK)
    out = _embed_gather(idx, weight)
    return out.reshape(b, l, _EMB)


# speedup vs baseline: 1.0424x; 1.0424x over previous
import functools

import jax
import jax.numpy as jnp
from jax import lax
from jax.experimental import pallas as pl
from jax.experimental.pallas import tpu as pltpu
from jax.experimental.pallas import tpu_sc as plsc

_EMB = 64
_N = 4096 * 50

_info = plsc.get_sparse_core_info()
_NC = _info.num_cores
_NS = _info.num_subcores
_NW = _NC * _NS
_BPW = _N // _NW
_CHUNK = 100
_NCH = _BPW // _CHUNK
_NBUF = 4
_QUADS = _NCH // _NBUF

_mesh = plsc.VectorSubcoreMesh(core_axis_name="c", subcore_axis_name="s")


@functools.partial(
    pl.kernel,
    mesh=_mesh,
    compiler_params=pltpu.CompilerParams(use_tc_tiling_on_sc=False),
    out_type=jax.ShapeDtypeStruct((_N, _EMB), jnp.float32),
    scratch_types=[
        pltpu.VMEM((_NCH, _CHUNK), jnp.int32),
    ]
    + [pltpu.VMEM((_CHUNK, _EMB), jnp.float32) for _ in range(_NBUF)]
    + [pltpu.SemaphoreType.DMA for _ in range(2 * _NBUF)],
)
def _embed_gather(idx_hbm, table_hbm, out_hbm, idx_v, *bufs_and_sems):
    rows = bufs_and_sems[:_NBUF]
    gs = bufs_and_sems[_NBUF : 2 * _NBUF]
    ws = bufs_and_sems[2 * _NBUF :]
    wid = lax.axis_index("s") * _NC + lax.axis_index("c")
    base = wid * _BPW
    pltpu.sync_copy(idx_hbm.at[wid], idx_v)

    for j in range(_NBUF):
        pltpu.async_copy(table_hbm.at[idx_v.at[j]], rows[j], gs[j])

    def out_slice(c):
        return out_hbm.at[pl.ds(base + c * _CHUNK, _CHUNK)]

    def quad(i, carry):
        for j in range(_NBUF):
            c = _NBUF * i + j
            pltpu.make_async_copy(table_hbm.at[idx_v.at[c]], rows[j], gs[j]).wait()
            pltpu.async_copy(rows[j], out_slice(c), ws[j])
        for j in range(_NBUF):
            c = _NBUF * i + j
            pltpu.make_async_copy(rows[j], out_slice(c), ws[j]).wait()
            pltpu.async_copy(table_hbm.at[idx_v.at[c + _NBUF]], rows[j], gs[j])
        return carry

    lax.fori_loop(0, _QUADS - 1, quad, 0)

    for j in range(_NBUF):
        c = _NBUF * (_QUADS - 1) + j
        pltpu.make_async_copy(table_hbm.at[idx_v.at[c]], rows[j], gs[j]).wait()
        pltpu.async_copy(rows[j], out_slice(c), ws[j])
    for j in range(_NBUF):
        c = _NBUF * (_QUADS - 1) + j
        pltpu.make_async_copy(rows[j], out_slice(c), ws[j]).wait()


def kernel(token_ids, weight):
    b, l = token_ids.shape
    idx = token_ids.astype(jnp.int32).reshape(_NW, _NCH, _CHUNK)
    out = _embed_gather(idx, weight)
    return out.reshape(b, l, _EMB)


# speedup vs baseline: 1.0444x; 1.0020x over previous
import functools

import jax
import jax.numpy as jnp
from jax import lax
from jax.experimental import pallas as pl
from jax.experimental.pallas import tpu as pltpu
from jax.experimental.pallas import tpu_sc as plsc

_EMB = 64
_PAD = 128
_N = 4096 * 50

_info = plsc.get_sparse_core_info()
_NC = _info.num_cores
_NS = _info.num_subcores
_NW = _NC * _NS

_BPW = _N // _NW
_CHUNK = 160
_NCH = _BPW // _CHUNK
_NBUF = 4
_QUADS = _NCH // _NBUF

_mesh = plsc.VectorSubcoreMesh(core_axis_name="c", subcore_axis_name="s")


@functools.partial(
    pl.kernel,
    mesh=_mesh,
    out_type=jax.ShapeDtypeStruct((_N, _PAD), jnp.float32),
    scratch_types=[
        pltpu.VMEM((_BPW,), jnp.int32),
    ]
    + [pltpu.VMEM((_CHUNK, _PAD), jnp.float32) for _ in range(_NBUF)]
    + [pltpu.SemaphoreType.DMA for _ in range(2 * _NBUF)],
)
def _embed_gather(idx_hbm, table_hbm, out_hbm, idx_v, *bufs_and_sems):
    rows = bufs_and_sems[:_NBUF]
    gs = bufs_and_sems[_NBUF : 2 * _NBUF]
    ws = bufs_and_sems[2 * _NBUF :]
    wid = lax.axis_index("s") * _NC + lax.axis_index("c")
    base = wid * _BPW
    pltpu.sync_copy(idx_hbm.at[pl.ds(base, _BPW)], idx_v)

    def idx_slice(c):
        return idx_v.at[pl.ds(c * _CHUNK, _CHUNK)]

    def out_slice(c):
        return out_hbm.at[pl.ds(base + c * _CHUNK, _CHUNK)]

    for j in range(_NBUF):
        pltpu.async_copy(table_hbm.at[idx_slice(j)], rows[j], gs[j])

    def quad(i, carry):
        for j in range(_NBUF):
            c = _NBUF * i + j
            pltpu.make_async_copy(table_hbm.at[idx_slice(c)], rows[j], gs[j]).wait()
            pltpu.async_copy(rows[j], out_slice(c), ws[j])
        for j in range(_NBUF):
            c = _NBUF * i + j
            pltpu.make_async_copy(rows[j], out_slice(c), ws[j]).wait()
            pltpu.async_copy(table_hbm.at[idx_slice(c + _NBUF)], rows[j], gs[j])
        return carry

    lax.fori_loop(0, _QUADS - 1, quad, 0)

    for j in range(_NBUF):
        c = _NBUF * (_QUADS - 1) + j
        pltpu.make_async_copy(table_hbm.at[idx_slice(c)], rows[j], gs[j]).wait()
        pltpu.async_copy(rows[j], out_slice(c), ws[j])
    for j in range(_NBUF):
        c = _NBUF * (_QUADS - 1) + j
        pltpu.make_async_copy(rows[j], out_slice(c), ws[j]).wait()


def kernel(token_ids, weight):
    b, l = token_ids.shape
    idx = token_ids.astype(jnp.int32).reshape(-1)
    wp = jnp.pad(weight, ((0, 0), (0, _PAD - _EMB)))
    out = _embed_gather(idx, wp)
    return out[:, :_EMB].reshape(b, l, _EMB)
